# baseline (speedup 1.0000x reference)
import functools

import jax
import jax.numpy as jnp
from jax.experimental import pallas as pl
from jax.experimental.pallas import tpu as pltpu

_MIB = 1024 * 1024

_IDS_PER_STEP = 512


def _round_up(x: int, m: int) -> int:
    return ((x + m - 1) // m) * m


def _row_gather_kernel(ids_ref, table_ref, out_ref, *, ips):
    base = pl.program_id(0) * ips
    table_3d = table_ref.reshape(table_ref.shape[0], 1, table_ref.shape[1])
    for k in range(ips):
        idx = ids_ref[base + k]
        out_ref[k, :] = table_3d[idx, 0]


def kernel(user_id: jax.Array, table: jax.Array) -> jax.Array:
    users_num, hidden = table.shape
    orig_shape = user_id.shape
    dtype = table.dtype

    flat_ids = user_id.reshape(-1).astype(jnp.int32)
    num_ids = flat_ids.shape[0]

    hidden_p = _round_up(hidden, 128)
    table_p = table
    if hidden_p != hidden:
        table_p = jnp.pad(table, ((0, 0), (0, hidden_p - hidden)))

    ips = min(_IDS_PER_STEP, _round_up(num_ids, 8))
    num_steps = pl.cdiv(num_ids, ips)
    padded = num_steps * ips
    if padded != num_ids:
        flat_ids = jnp.pad(flat_ids, (0, padded - num_ids))

    out_shape = jax.ShapeDtypeStruct((padded, hidden_p), dtype)
    itemsize = jnp.dtype(dtype).itemsize
    table_bytes = users_num * hidden_p * itemsize
    vmem_limit = int(min(56 * _MIB,
                         2 * table_bytes + 4 * ips * hidden_p * itemsize
                         + 8 * _MIB))
    compiler_params = pltpu.CompilerParams(
        dimension_semantics=("parallel",),
        vmem_limit_bytes=vmem_limit)
    body = functools.partial(_row_gather_kernel, ips=ips)

    def build(single_buffer_table: bool):
        table_kwargs = {}
        if single_buffer_table:
            table_kwargs["pipeline_mode"] = pl.Buffered(1)
        grid_spec = pltpu.PrefetchScalarGridSpec(
            num_scalar_prefetch=1,
            grid=(num_steps,),
            in_specs=[
                pl.BlockSpec((users_num, hidden_p),
                             lambda i, ids: (0, 0), **table_kwargs),
            ],
            out_specs=pl.BlockSpec((ips, hidden_p), lambda i, ids: (i, 0)),
        )
        return pl.pallas_call(body, grid_spec=grid_spec,
                              out_shape=out_shape,
                              compiler_params=compiler_params)

    try:
        out = build(single_buffer_table=True)(flat_ids, table_p)
    except Exception:
        out = build(single_buffer_table=False)(flat_ids, table_p)

    out = out[:num_ids, :hidden]
    return out.reshape(orig_shape + (hidden,))


# speedup vs baseline: 1.0003x; 1.0003x over previous
import functools

import jax
import jax.numpy as jnp
from jax.experimental import pallas as pl
from jax.experimental.pallas import tpu as pltpu

_MIB = 1024 * 1024

_IDS_PER_STEP = 512


def _round_up(x: int, m: int) -> int:
    return ((x + m - 1) // m) * m


def _row_gather_kernel(ids_ref, table_ref, out_ref, *, ips):
    base = pl.program_id(0) * ips
    for k in range(ips):
        idx = ids_ref[base + k]
        out_ref[k, :] = table_ref[idx, 0]


def kernel(user_id: jax.Array, table: jax.Array) -> jax.Array:
    users_num, hidden = table.shape
    orig_shape = user_id.shape
    dtype = table.dtype

    flat_ids = user_id.reshape(-1).astype(jnp.int32)
    num_ids = flat_ids.shape[0]

    hidden_p = _round_up(hidden, 128)
    table_p = table
    if hidden_p != hidden:
        table_p = jnp.pad(table, ((0, 0), (0, hidden_p - hidden)))
    table_3d = table_p.reshape(users_num, 1, hidden_p)

    ips = min(_IDS_PER_STEP, _round_up(num_ids, 8))
    num_steps = pl.cdiv(num_ids, ips)
    padded = num_steps * ips
    if padded != num_ids:
        flat_ids = jnp.pad(flat_ids, (0, padded - num_ids))

    out_shape = jax.ShapeDtypeStruct((padded, hidden_p), dtype)
    itemsize = jnp.dtype(dtype).itemsize
    table_bytes = users_num * hidden_p * itemsize
    vmem_limit = int(min(56 * _MIB,
                         2 * table_bytes + 4 * ips * hidden_p * itemsize
                         + 8 * _MIB))
    compiler_params = pltpu.CompilerParams(
        dimension_semantics=("parallel",),
        vmem_limit_bytes=vmem_limit)
    body = functools.partial(_row_gather_kernel, ips=ips)

    def build(single_buffer_table: bool):
        table_kwargs = {}
        if single_buffer_table:
            table_kwargs["pipeline_mode"] = pl.Buffered(1)
        grid_spec = pltpu.PrefetchScalarGridSpec(
            num_scalar_prefetch=1,
            grid=(num_steps,),
            in_specs=[
                pl.BlockSpec((users_num, 1, hidden_p),
                             lambda i, ids: (0, 0, 0), **table_kwargs),
            ],
            out_specs=pl.BlockSpec((ips, hidden_p), lambda i, ids: (i, 0)),
        )
        return pl.pallas_call(body, grid_spec=grid_spec,
                              out_shape=out_shape,
                              compiler_params=compiler_params)

    try:
        out = build(single_buffer_table=True)(flat_ids, table_3d)
    except Exception:
        out = build(single_buffer_table=False)(flat_ids, table_3d)

    out = out[:num_ids, :hidden]
    return out.reshape(orig_shape + (hidden,))
